# baseline (speedup 1.0000x reference)
import functools

import jax
import jax.numpy as jnp
from jax import lax
from jax.experimental import pallas as pl
from jax.experimental.pallas import tpu as pltpu, tpu_sc as plsc

_NC = 2
_NS = 16
_NW = _NC * _NS
_L = 16

_BATCH = 16384
_DIM = 32
_BW = _BATCH // _NW
_CH = 128
_NCHUNK = _BW // _CH


def _rec_mf_body(users_hbm, items_hbm, u_tab_hbm, i_tab_hbm, out_hbm,
                 idx_u, idx_i, u_rows, i_rows, out_v, sem):
    wid = lax.axis_index("s") * _NC + lax.axis_index("c")
    base = wid * _BW

    pltpu.sync_copy(users_hbm.at[pl.ds(base, _BW)], idx_u)
    pltpu.sync_copy(items_hbm.at[pl.ds(base, _BW)], idx_i)

    copies = []
    for c in range(_NCHUNK):
        sl = pl.ds(c * _CH, _CH)
        copies.append(pltpu.async_copy(
            u_tab_hbm.at[idx_u.at[sl]], u_rows.at[sl], sem))
        copies.append(pltpu.async_copy(
            i_tab_hbm.at[idx_i.at[sl]], i_rows.at[sl], sem))
    for cp in copies:
        cp.wait()

    lane_iota = lax.iota(jnp.int32, _L)

    def tile_body(t, _):
        row0 = t * _L
        acc = jnp.zeros((_L,), jnp.float32)
        for r in range(_L):
            row = row0 + r
            s = (u_rows[row, pl.ds(0, _L)] * i_rows[row, pl.ds(0, _L)]
                 + u_rows[row, pl.ds(_L, _L)] * i_rows[row, pl.ds(_L, _L)])
            acc = acc + jnp.where(lane_iota == r, jnp.sum(s, axis=0), 0.0)
        out_v[pl.ds(row0, _L)] = 1.0 / (1.0 + jnp.exp(-acc))
        return 0

    lax.fori_loop(0, _BW // _L, tile_body, 0)

    pltpu.sync_copy(out_v, out_hbm.at[pl.ds(base, _BW)])


@jax.jit
def kernel(users, items, user_table, item_table):
    mesh = plsc.VectorSubcoreMesh(
        core_axis_name="c", subcore_axis_name="s",
        num_cores=_NC, num_subcores=_NS)
    f = pl.kernel(
        _rec_mf_body,
        out_type=jax.ShapeDtypeStruct((_BATCH,), jnp.float32),
        mesh=mesh,
        compiler_params=pltpu.CompilerParams(
            needs_layout_passes=False, use_tc_tiling_on_sc=False),
        scratch_types=[
            pltpu.VMEM((_BW,), jnp.int32),
            pltpu.VMEM((_BW,), jnp.int32),
            pltpu.VMEM((_BW, _DIM), jnp.float32),
            pltpu.VMEM((_BW, _DIM), jnp.float32),
            pltpu.VMEM((_BW,), jnp.float32),
            pltpu.SemaphoreType.DMA,
        ],
    )
    return f(users, items, user_table, item_table)
